# Initial kernel scaffold; baseline (speedup 1.0000x reference)
import jax
import jax.numpy as jnp
from jax.experimental import pallas as pl


def kernel(h_frontier, neighbor_ids, rel_ids, type_ids, frontier_node_ids, type_emb, rel_emb, W_ih, W_hh, b_ih, b_hh, exp_W1, exp_b1, exp_w2, exp_b2, nbr_W1, nbr_b1, nbr_w2, nbr_b2):
    raise NotImplementedError("write your pallas kernel here")


# speedup vs baseline: 1.6233x; 1.6233x over previous
import functools

import jax
import jax.numpy as jnp
from jax.experimental import pallas as pl
from jax.experimental.pallas import tpu as pltpu

HIDDEN = 128
NUM_NODES = 100000
F = 8192
DEG = 32
M = 2048
K = 16

ALPHA_BLK = 1024
EXP_BLK = 128
MSG_TOTAL = M * K + M
MSG_CHUNK = 2048


def _mm_nt(a, b):
    return jax.lax.dot_general(a.astype(jnp.bfloat16), b.astype(jnp.bfloat16),
                               (((1,), (1,)), ((), ())),
                               preferred_element_type=jnp.float32)


def _alpha_kernel(h_ref, w1_ref, b1_ref, w2_ref, alpha_ref):
    hid = _mm_nt(h_ref[...], w1_ref[...])
    hid = jnp.maximum(hid + b1_ref[...][None, :], 0.0)
    w2 = w2_ref[...].astype(jnp.bfloat16).astype(jnp.float32)
    hid = hid.astype(jnp.bfloat16).astype(jnp.float32)
    alpha_ref[...] = jnp.sum(hid * w2[None, :], axis=1)


def _expander_kernel(hE_ref, relE_ref, typeE_ref, nbrE_ref,
                     rel_emb_ref, type_emb_ref,
                     w_ih_ref, w_hh_ref, b_ih_ref, b_hh_ref,
                     nbr_w1_ref, nbr_b1_ref, nbr_w2_ref,
                     msgs_ref, dst_ref):
    blk = EXP_BLK
    E = blk * DEG
    f32 = jnp.float32

    w_r = w_ih_ref[:, :HIDDEN]
    w_t = w_ih_ref[:, HIDDEN:]
    rel_gi = _mm_nt(rel_emb_ref[...], w_r)
    type_gi = _mm_nt(type_emb_ref[...], w_t)

    hE = hE_ref[...]
    gh = _mm_nt(hE, w_hh_ref[...])
    gh = gh + b_hh_ref[...][None, :]

    relE = relE_ref[...]
    relE3 = relE[:, :, None]
    gi_r = jnp.zeros((blk, DEG, 3 * HIDDEN), f32)
    for r in range(12):
        gi_r = jnp.where(relE3 == r, rel_gi[r][None, None, :], gi_r)
    gi_r = gi_r.reshape(E, 3 * HIDDEN)

    typeE = typeE_ref[...]
    typeE2 = typeE[:, None]
    gi_t = jnp.zeros((blk, 3 * HIDDEN), f32)
    for t in range(4):
        gi_t = jnp.where(typeE2 == t, type_gi[t][None, :], gi_t)
    gi_t = gi_t + b_ih_ref[...][None, :]

    def rep(x):
        c = x.shape[1]
        return jnp.broadcast_to(x[:, None, :], (blk, DEG, c)).reshape(E, c)

    gi = gi_r + rep(gi_t)
    ghe = rep(gh)
    h_rep = rep(hE)

    r = jax.nn.sigmoid(gi[:, :HIDDEN] + ghe[:, :HIDDEN])
    z = jax.nn.sigmoid(gi[:, HIDDEN:2 * HIDDEN] + ghe[:, HIDDEN:2 * HIDDEN])
    n = jnp.tanh(gi[:, 2 * HIDDEN:] + r * ghe[:, 2 * HIDDEN:])
    g = (1.0 - z) * n + z * h_rep

    hid = _mm_nt(g, nbr_w1_ref[...])
    hid = jnp.maximum(hid + nbr_b1_ref[...][None, :], 0.0)
    hid = hid.astype(jnp.bfloat16).astype(f32)
    w2 = nbr_w2_ref[...].astype(jnp.bfloat16).astype(f32)
    beta = jnp.sum(hid.reshape(blk, DEG, HIDDEN) *
                   w2[None, None, :], axis=2)

    g3 = g.reshape(blk, DEG, HIDDEN)
    nbrE = nbrE_ref[...]
    iota_d = jax.lax.broadcasted_iota(jnp.int32, (blk, DEG), 1)
    neg_inf = jnp.float32(-jnp.inf)
    for k in range(K):
        mv = jnp.max(beta, axis=1, keepdims=True)
        is_max = beta >= mv
        fidx = jnp.min(jnp.where(is_max, iota_d, DEG), axis=1, keepdims=True)
        oh = iota_d == fidx
        g_sel = jnp.sum(g3 * oh.astype(f32)[:, :, None], axis=1)
        dst = jnp.sum(jnp.where(oh, nbrE, 0), axis=1, keepdims=True)
        msgs_ref[:, k, :] = g_sel
        dst_ref[:, k:k + 1] = dst
        beta = jnp.where(oh, neg_inf, beta)


def _scatter_kernel(seg_ref, msgs_ref, out_ref):
    m = pl.program_id(0)

    @pl.when(m == 0)
    def _():
        out_ref[...] = jnp.zeros_like(out_ref)

    def body(i, _):
        d = seg_ref[i]
        row = jnp.exp(msgs_ref[pl.ds(i, 1), :])
        out_ref[pl.ds(d, 1), :] += row
        return 0

    jax.lax.fori_loop(0, MSG_CHUNK, body, 0)

    @pl.when(m == pl.num_programs(0) - 1)
    def _():
        acc = out_ref[...]
        out_ref[...] = jnp.where(acc > 0.0, jnp.log(acc), 0.0)


@jax.jit
def kernel(h_frontier, neighbor_ids, rel_ids, type_ids, frontier_node_ids,
           type_emb, rel_emb, W_ih, W_hh, b_ih, b_hh,
           exp_W1, exp_b1, exp_w2, exp_b2,
           nbr_W1, nbr_b1, nbr_w2, nbr_b2):
    f32 = jnp.float32

    alpha = pl.pallas_call(
        _alpha_kernel,
        grid=(F // ALPHA_BLK,),
        in_specs=[
            pl.BlockSpec((ALPHA_BLK, HIDDEN), lambda i: (i, 0)),
            pl.BlockSpec((HIDDEN, HIDDEN), lambda i: (0, 0)),
            pl.BlockSpec((HIDDEN,), lambda i: (0,)),
            pl.BlockSpec((HIDDEN,), lambda i: (0,)),
        ],
        out_specs=pl.BlockSpec((ALPHA_BLK,), lambda i: (i,)),
        out_shape=jax.ShapeDtypeStruct((F,), f32),
    )(h_frontier, exp_W1, exp_b1, exp_w2)

    _, top_idx = jax.lax.top_k(alpha, M)
    hE = jnp.take(h_frontier, top_idx, axis=0)
    nbrE = jnp.take(neighbor_ids, top_idx, axis=0)
    relE = jnp.take(rel_ids, top_idx, axis=0)
    typeE = jnp.take(type_ids, top_idx, axis=0)
    nodeE = jnp.take(frontier_node_ids, top_idx, axis=0)

    msgs, dst = pl.pallas_call(
        _expander_kernel,
        grid=(M // EXP_BLK,),
        in_specs=[
            pl.BlockSpec((EXP_BLK, HIDDEN), lambda i: (i, 0)),
            pl.BlockSpec((EXP_BLK, DEG), lambda i: (i, 0)),
            pl.BlockSpec((EXP_BLK,), lambda i: (i,)),
            pl.BlockSpec((EXP_BLK, DEG), lambda i: (i, 0)),
            pl.BlockSpec((12, HIDDEN), lambda i: (0, 0)),
            pl.BlockSpec((4, HIDDEN), lambda i: (0, 0)),
            pl.BlockSpec((3 * HIDDEN, 2 * HIDDEN), lambda i: (0, 0)),
            pl.BlockSpec((3 * HIDDEN, HIDDEN), lambda i: (0, 0)),
            pl.BlockSpec((3 * HIDDEN,), lambda i: (0,)),
            pl.BlockSpec((3 * HIDDEN,), lambda i: (0,)),
            pl.BlockSpec((HIDDEN, HIDDEN), lambda i: (0, 0)),
            pl.BlockSpec((HIDDEN,), lambda i: (0,)),
            pl.BlockSpec((HIDDEN,), lambda i: (0,)),
        ],
        out_specs=[
            pl.BlockSpec((EXP_BLK, K, HIDDEN), lambda i: (i, 0, 0)),
            pl.BlockSpec((EXP_BLK, K), lambda i: (i, 0)),
        ],
        out_shape=[
            jax.ShapeDtypeStruct((M, K, HIDDEN), f32),
            jax.ShapeDtypeStruct((M, K), jnp.int32),
        ],
    )(hE, relE, typeE, nbrE, rel_emb, type_emb, W_ih, W_hh, b_ih, b_hh,
      nbr_W1, nbr_b1, nbr_w2)

    all_msgs = jnp.concatenate([msgs.reshape(M * K, HIDDEN), hE], axis=0)
    seg = jnp.concatenate([dst.reshape(-1), nodeE], axis=0)

    out = pl.pallas_call(
        _scatter_kernel,
        grid=(MSG_TOTAL // MSG_CHUNK,),
        in_specs=[
            pl.BlockSpec((MSG_CHUNK,), lambda i: (i,),
                         memory_space=pltpu.SMEM),
            pl.BlockSpec((MSG_CHUNK, HIDDEN), lambda i: (i, 0)),
        ],
        out_specs=pl.BlockSpec((NUM_NODES, HIDDEN), lambda i: (0, 0)),
        out_shape=jax.ShapeDtypeStruct((NUM_NODES, HIDDEN), f32),
    )(seg, all_msgs)
    return out
